# Initial kernel scaffold; baseline (speedup 1.0000x reference)
import jax
import jax.numpy as jnp
from jax.experimental import pallas as pl


def kernel(text_input, attention_mask, image_input, audio_input, text_emb, W_img, b_img, W_aud, b_aud, W_r, b_r, W_e, b_e, W_h, b_h):
    raise NotImplementedError("write your pallas kernel here")


# speedup vs baseline: 2.0948x; 2.0948x over previous
import functools

import jax
import jax.numpy as jnp
from jax import lax
from jax.experimental import pallas as pl
from jax.experimental.pallas import tpu as pltpu
from jax.experimental.pallas import tpu_sc as plsc

_F32 = jnp.float32


def _sc_gather(table, ids):
    n = ids.shape[0]
    d = table.shape[1]
    info = plsc.get_sparse_core_info()
    nw = info.num_cores * info.num_subcores
    bpw = n // nw
    mesh = plsc.VectorSubcoreMesh(core_axis_name="c", subcore_axis_name="s")

    @functools.partial(
        pl.kernel,
        mesh=mesh,
        out_type=jax.ShapeDtypeStruct((n, d), _F32),
        scratch_types=[
            pltpu.VMEM((bpw,), jnp.int32),
            pltpu.VMEM((bpw, d), _F32),
            pltpu.SemaphoreType.DMA,
        ],
    )
    def gather_kernel(table_hbm, idx_hbm, out_hbm, idx_v, rows_v, sem):
        wid = lax.axis_index("s") * info.num_cores + lax.axis_index("c")
        base = wid * bpw
        pltpu.sync_copy(idx_hbm.at[pl.ds(base, bpw)], idx_v)
        pltpu.async_copy(table_hbm.at[idx_v], rows_v, sem).wait()
        pltpu.sync_copy(rows_v, out_hbm.at[pl.ds(base, bpw)])

    return gather_kernel(table, ids)


def _softmax_rows(logits):
    m = jnp.max(logits, axis=1, keepdims=True)
    p = jnp.exp(logits - m)
    return p / jnp.sum(p, axis=1, keepdims=True)


def _reduce_body(txt_ref, mask_ref, img_ref, aud_ref, wi_ref, bi_ref,
                 wa_ref, ba_ref, wr_ref, br_ref, a_ref, g_ref):
    txt = txt_ref[0] * mask_ref[0]
    img = jnp.dot(img_ref[0], wi_ref[...],
                  preferred_element_type=_F32) + bi_ref[...]
    aud = jnp.dot(aud_ref[0], wa_ref[...],
                  preferred_element_type=_F32) + ba_ref[...]
    br = br_ref[...]
    gt = _softmax_rows(jnp.dot(txt, wr_ref[...], preferred_element_type=_F32) + br)
    gi = _softmax_rows(jnp.dot(img, wr_ref[...], preferred_element_type=_F32) + br)
    ga = _softmax_rows(jnp.dot(aud, wr_ref[...], preferred_element_type=_F32) + br)
    contract = (((0,), (0,)), ((), ()))
    a = (lax.dot_general(gt, txt, contract, preferred_element_type=_F32)
         + lax.dot_general(gi, img, contract, preferred_element_type=_F32)
         + lax.dot_general(ga, aud, contract, preferred_element_type=_F32))
    a_ref[0] = a
    g_ref[0] = (jnp.sum(gt, axis=0, keepdims=True)
                + jnp.sum(gi, axis=0, keepdims=True)
                + jnp.sum(ga, axis=0, keepdims=True))


def _gate_reduce(txt, maskf, imgp, audp, w_img, b_img2, w_aud, b_aud2, w_r, b_r2):
    b, s, d = txt.shape
    np_ = imgp.shape[1]
    af, al = audp.shape[1], audp.shape[2]
    e = w_r.shape[1]
    full = lambda shp: pl.BlockSpec(shp, lambda i: (0,) * len(shp))
    return pl.pallas_call(
        _reduce_body,
        grid=(b,),
        in_specs=[
            pl.BlockSpec((1, s, d), lambda i: (i, 0, 0)),
            pl.BlockSpec((1, s, 1), lambda i: (i, 0, 0)),
            pl.BlockSpec((1, np_, 768), lambda i: (i, 0, 0)),
            pl.BlockSpec((1, af, al), lambda i: (i, 0, 0)),
            full((768, d)),
            full((1, d)),
            full((al, d)),
            full((1, d)),
            full((d, e)),
            full((1, e)),
        ],
        out_specs=[
            pl.BlockSpec((1, e, d), lambda i: (i, 0, 0)),
            pl.BlockSpec((1, 1, e), lambda i: (i, 0, 0)),
        ],
        out_shape=[
            jax.ShapeDtypeStruct((b, e, d), _F32),
            jax.ShapeDtypeStruct((b, 1, e), _F32),
        ],
    )(txt, maskf, imgp, audp, w_img, b_img2, w_aud, b_aud2, w_r, b_r2)


def _finish_body(a_ref, g_ref, we_ref, be_ref, wh_ref, bh_ref, out_ref, *,
                 n_experts, inv_t):
    g = g_ref[:, 0, :]
    pooled = jnp.dot(g, be_ref[...], preferred_element_type=_F32)
    for e in range(n_experts):
        pooled = pooled + jnp.dot(a_ref[:, e, :], we_ref[e],
                                  preferred_element_type=_F32)
    pooled = pooled * inv_t
    out_ref[...] = jnp.dot(pooled, wh_ref[...],
                           preferred_element_type=_F32) + bh_ref[...]


def _finish(a, g, w_e, b_e, w_h, b_h2, n_tokens):
    b = a.shape[0]
    e, d, _ = w_e.shape
    c = w_h.shape[1]
    body = functools.partial(_finish_body, n_experts=e, inv_t=1.0 / n_tokens)
    return pl.pallas_call(
        body,
        out_shape=jax.ShapeDtypeStruct((b, c), _F32),
    )(a, g, w_e, b_e, w_h, b_h2)


def kernel(text_input, attention_mask, image_input, audio_input, text_emb,
           W_img, b_img, W_aud, b_aud, W_r, b_r, W_e, b_e, W_h, b_h):
    b, s = text_input.shape
    v, d = text_emb.shape
    p = 16
    np_ = (224 // p) * (224 // p)
    af = 100
    al = audio_input.shape[1] // af
    n_tokens = s + np_ + af

    ids = text_input.reshape(-1).astype(jnp.int32)
    maskf = attention_mask.astype(_F32).reshape(b, s, 1)
    imgp = image_input.reshape(b, 3, 224 // p, p, 224 // p, p)
    imgp = imgp.transpose(0, 2, 4, 1, 3, 5).reshape(b, np_, 3 * p * p)
    audp = audio_input.reshape(b, af, al)
    b_img2 = b_img.reshape(1, d)
    b_aud2 = b_aud.reshape(1, d)
    b_r2 = b_r.reshape(1, -1)
    b_h2 = b_h.reshape(1, -1)

    txt = _sc_gather(text_emb, ids).reshape(b, s, d)

    a, g = _gate_reduce(txt, maskf, imgp, audp, W_img, b_img2, W_aud,
                        b_aud2, W_r, b_r2)
    return _finish(a, g, W_e, b_e, W_h, b_h2, n_tokens)
